# baseline (speedup 1.0000x reference)
import dataclasses

import jax
import jax.numpy as jnp
from jax import lax
from jax.experimental import pallas as pl
from jax.experimental.pallas import tpu as pltpu
from jax.experimental.pallas import tpu_sc as plsc

_NW = 32
_LANES = 16


def _make_body(B, dim, H, W, M):
    BPW = B // _NW
    PP = BPW * M
    EPW = PP * dim
    NCH = EPW // _LANES
    NI = B * M
    assert dim == 2 and BPW == 2 and EPW % _LANES == 0
    assert W & (W - 1) == 0
    WSH = W.bit_length() - 1
    SHB = (H * W - 1).bit_length()

    def body(feat_ref, pk_ref, out_ref,
             cmb_v, feat_v, t01_v, acc_v,
             sem_f, sem_t, sem_u):
        wid = lax.axis_index("s") * 2 + lax.axis_index("c")
        feat_cp = pltpu.async_copy(
            feat_ref.at[pl.ds(wid * BPW, BPW)], feat_v, sem_f)
        t0_cp = pltpu.async_copy(
            pk_ref.at[pl.ds(NI + wid * PP, PP)], t01_v.at[pl.ds(0, PP)],
            sem_t)
        t1_cp = pltpu.async_copy(
            pk_ref.at[pl.ds(2 * NI + wid * PP, PP)], t01_v.at[pl.ds(PP, PP)],
            sem_u)
        pltpu.sync_copy(pk_ref.at[pl.ds(wid * PP, PP)], cmb_v)

        iota = lax.broadcasted_iota(jnp.int32, (_LANES,), 0)
        half = lax.shift_right_logical(iota, 1)
        d_vec = iota & 1
        t0_cp.wait()
        t1_cp.wait()
        feat_cp.wait()

        acc_l = jnp.zeros((_LANES,), jnp.float32)
        acc_m = jnp.zeros((_LANES,), jnp.float32)
        for c in range(NCH):
            p_rel = half + (c * (_LANES // dim))
            b_loc = jnp.where(p_rel >= M, 1, 0)
            cmb_g = plsc.load_gather(cmb_v, [p_rel])
            ind_g = cmb_g & (H * W - 1)
            v = plsc.load_gather(
                feat_v,
                [b_loc, d_vec, lax.shift_right_logical(ind_g, WSH),
                 ind_g & (W - 1)])
            t = plsc.bitcast(
                plsc.load_gather(t01_v, [p_rel + d_vec * PP]), jnp.float32)
            m = lax.shift_right_logical(cmb_g, SHB).astype(jnp.float32)
            a = jnp.abs(v - t) * m
            acc_l = acc_l + jnp.where(a < 1.0, 0.5 * a * a, a - 0.5)
            acc_m = acc_m + m
        acc_v[0, :] = acc_l
        acc_v[1, :] = acc_m
        pltpu.sync_copy(acc_v, out_ref.at[wid])

    return body


def kernel(output, mask, ind, target):
    B, dim, H, W = output.shape
    M = ind.shape[1]
    BPW = B // _NW
    PP = BPW * M

    packed = jnp.concatenate([
        (ind | (mask << (H * W - 1).bit_length())).reshape(-1),
        lax.bitcast_convert_type(target[:, :, 0], jnp.int32).reshape(-1),
        lax.bitcast_convert_type(target[:, :, 1], jnp.int32).reshape(-1),
    ])

    cp = pltpu.CompilerParams()
    if "needs_layout_passes" in pltpu.CompilerParams.__dataclass_fields__:
        cp = dataclasses.replace(cp, needs_layout_passes=False)
    mesh = plsc.VectorSubcoreMesh(core_axis_name="c", subcore_axis_name="s")
    fn = pl.kernel(
        _make_body(B, dim, H, W, M),
        out_type=jax.ShapeDtypeStruct((_NW, 2, _LANES), jnp.float32),
        mesh=mesh,
        compiler_params=cp,
        scratch_types=[
            pltpu.VMEM((PP,), jnp.int32),
            pltpu.VMEM((BPW, dim, H, W), jnp.float32),
            pltpu.VMEM((2 * PP,), jnp.int32),
            pltpu.VMEM((2, _LANES), jnp.float32),
            pltpu.SemaphoreType.DMA,
            pltpu.SemaphoreType.DMA,
            pltpu.SemaphoreType.DMA,
        ],
    )
    parts = fn(output, packed)
    loss = parts[:, 0, :].sum()
    num = parts[:, 1, :].sum() / dim
    return loss / (num + 1e-4)


# speedup vs baseline: 1.1874x; 1.1874x over previous
import dataclasses

import jax
import jax.numpy as jnp
from jax import lax
from jax.experimental import pallas as pl
from jax.experimental.pallas import tpu as pltpu
from jax.experimental.pallas import tpu_sc as plsc

_NW = 32
_LANES = 16


def _make_body(B, dim, H, W, M):
    BPW = B // _NW
    PP = BPW * M
    EPW = PP * dim
    NCH = EPW // _LANES
    NI = B * M
    assert dim == 2 and BPW == 2 and EPW % _LANES == 0
    assert W & (W - 1) == 0
    WSH = W.bit_length() - 1

    def body(feat_ref, pk_ref, out_ref,
             ind_v, msk_v, feat_v, t0_v, t1_v, acc_v,
             sem_f, sem_t, sem_u, sem_m):
        wid = lax.axis_index("s") * 2 + lax.axis_index("c")
        feat_cp = pltpu.async_copy(
            feat_ref.at[pl.ds(wid * BPW, BPW)], feat_v, sem_f)
        t0_cp = pltpu.async_copy(
            pk_ref.at[pl.ds(2 * NI + wid * PP, PP)], t0_v, sem_t)
        t1_cp = pltpu.async_copy(
            pk_ref.at[pl.ds(3 * NI + wid * PP, PP)], t1_v, sem_u)
        msk_cp = pltpu.async_copy(
            pk_ref.at[pl.ds(NI + wid * PP, PP)], msk_v, sem_m)
        pltpu.sync_copy(pk_ref.at[pl.ds(wid * PP, PP)], ind_v)

        iota = lax.broadcasted_iota(jnp.int32, (_LANES,), 0)
        half = lax.shift_right_logical(iota, 1)
        d_vec = iota & 1
        t0_cp.wait()
        t1_cp.wait()
        msk_cp.wait()
        feat_cp.wait()

        acc_l = jnp.zeros((_LANES,), jnp.float32)
        acc_m = jnp.zeros((_LANES,), jnp.float32)
        for c in range(NCH):
            p_rel = half + (c * (_LANES // dim))
            b_loc = jnp.where(p_rel >= M, 1, 0)
            ind_g = plsc.load_gather(ind_v, [p_rel])
            v = plsc.load_gather(
                feat_v,
                [b_loc, d_vec, lax.shift_right_logical(ind_g, WSH),
                 ind_g & (W - 1)])
            t0 = plsc.bitcast(plsc.load_gather(t0_v, [p_rel]), jnp.float32)
            t1 = plsc.bitcast(plsc.load_gather(t1_v, [p_rel]), jnp.float32)
            t = jnp.where(d_vec == 0, t0, t1)
            m = plsc.load_gather(msk_v, [p_rel]).astype(jnp.float32)
            a = jnp.abs(v - t) * m
            acc_l = acc_l + jnp.where(a < 1.0, 0.5 * a * a, a - 0.5)
            acc_m = acc_m + m
        acc_v[0, :] = acc_l
        acc_v[1, :] = acc_m
        pltpu.sync_copy(acc_v, out_ref.at[wid])

    return body


def kernel(output, mask, ind, target):
    B, dim, H, W = output.shape
    M = ind.shape[1]
    BPW = B // _NW
    PP = BPW * M

    packed = jnp.concatenate([
        ind.reshape(-1),
        mask.reshape(-1),
        lax.bitcast_convert_type(target[:, :, 0], jnp.int32).reshape(-1),
        lax.bitcast_convert_type(target[:, :, 1], jnp.int32).reshape(-1),
    ])

    cp = pltpu.CompilerParams()
    if "needs_layout_passes" in pltpu.CompilerParams.__dataclass_fields__:
        cp = dataclasses.replace(cp, needs_layout_passes=False)
    mesh = plsc.VectorSubcoreMesh(core_axis_name="c", subcore_axis_name="s")
    fn = pl.kernel(
        _make_body(B, dim, H, W, M),
        out_type=jax.ShapeDtypeStruct((_NW, 2, _LANES), jnp.float32),
        mesh=mesh,
        compiler_params=cp,
        scratch_types=[
            pltpu.VMEM((PP,), jnp.int32),
            pltpu.VMEM((PP,), jnp.int32),
            pltpu.VMEM((BPW, dim, H, W), jnp.float32),
            pltpu.VMEM((PP,), jnp.int32),
            pltpu.VMEM((PP,), jnp.int32),
            pltpu.VMEM((2, _LANES), jnp.float32),
            pltpu.SemaphoreType.DMA,
            pltpu.SemaphoreType.DMA,
            pltpu.SemaphoreType.DMA,
            pltpu.SemaphoreType.DMA,
        ],
    )
    parts = fn(output, packed)
    loss = parts[:, 0, :].sum()
    num = parts[:, 1, :].sum() / dim
    return loss / (num + 1e-4)
